# Initial kernel scaffold; baseline (speedup 1.0000x reference)
import jax
import jax.numpy as jnp
from jax.experimental import pallas as pl


def kernel(feature, edge_index, alpha, beta, Wg1, bg1, Wg2, bg2, Wf1, bf1, Wf2, bf2, Wm1, bm1, Wm2, bm2, bn_gamma, bn_beta):
    raise NotImplementedError("write your pallas kernel here")


# speedup vs baseline: 3.0041x; 3.0041x over previous
import functools

import jax
import jax.numpy as jnp
from jax import lax
from jax.experimental import pallas as pl
from jax.experimental.pallas import tpu as pltpu

_N = 10000
_E = 320000
_D = 128


def _k1_body(f_ref, s_ref, degp_ref, wg1_ref, bg1_ref, wf1_ref, bf1_ref,
             wf2_ref, bf2_ref, wm1_ref, bm1_ref, wm2_ref, bm2_ref,
             g_ref, b_ref, hw1_ref, z2_ref, dinv_ref):
    deg = degp_ref[0] + degp_ref[1] + 1.0
    dinv = lax.rsqrt(deg)
    dinv_ref[...] = dinv
    f = f_ref[...]
    hw1 = jnp.dot(f, wg1_ref[...], preferred_element_type=jnp.float32) + bg1_ref[...]
    hw1_ref[...] = hw1 * dinv
    t = jnp.dot(f, wf1_ref[...], preferred_element_type=jnp.float32) + bf1_ref[...]
    t = jnp.where(t > 0, t, jnp.exp(jnp.minimum(t, 0.0)) - 1.0)
    zf2 = jnp.dot(t, wf2_ref[...], preferred_element_type=jnp.float32) + bf2_ref[...]
    m = jnp.maximum(
        jnp.dot(s_ref[...], wm1_ref[...], preferred_element_type=jnp.float32)
        + bm1_ref[...], 0.0)
    m = jnp.dot(m, wm2_ref[...], preferred_element_type=jnp.float32) + bm2_ref[...]
    mu = jnp.mean(m)
    var = jnp.mean((m - mu) ** 2)
    mh = (m - mu) * lax.rsqrt(var + 1e-5) * g_ref[0, 0] + b_ref[0, 0]
    z2_ref[...] = zf2 * (1.0 / (1.0 + jnp.exp(-mh)))


def _k3_body(p_ref, hw1_ref, dinv_ref, wg2_ref, bg2_ref, hw2_ref):
    dinv = dinv_ref[...]
    h = jnp.maximum((p_ref[0] + p_ref[1] + hw1_ref[...]) * dinv, 0.0)
    hw2_ref[...] = (jnp.dot(h, wg2_ref[...], preferred_element_type=jnp.float32)
                    + bg2_ref[...]) * dinv


def _k5_body(q_ref, hw2_ref, dinv_ref, z1_ref):
    z1_ref[...] = (q_ref[0] + q_ref[1] + hw2_ref[...]) * dinv_ref[...]


def _deg_partials(dst):
    d0 = jnp.zeros((_N, 1), jnp.float32).at[dst, 0].add(1.0)
    return jnp.stack([d0, jnp.zeros_like(d0)])


def _scatter_partials(rows, src, dst):
    p0 = jnp.zeros((_N, _D), jnp.float32).at[dst].add(rows[src])
    return jnp.stack([p0, jnp.zeros_like(p0)])


def kernel(feature, edge_index, alpha, beta, Wg1, bg1, Wg2, bg2,
           Wf1, bf1, Wf2, bf2, Wm1, bm1, Wm2, bm2, bn_gamma, bn_beta):
    src = edge_index[0]
    dst = edge_index[1]
    s = jnp.stack([alpha, beta], axis=1)
    bg1r = bg1.reshape(1, -1)
    bg2r = bg2.reshape(1, -1)
    bf1r = bf1.reshape(1, -1)
    bf2r = bf2.reshape(1, -1)
    bm1r = bm1.reshape(1, -1)
    bm2r = bm2.reshape(1, -1)
    gr = bn_gamma.reshape(1, 1)
    br = bn_beta.reshape(1, 1)

    degp = _deg_partials(dst)

    hw1p, z2, dinv = pl.pallas_call(
        _k1_body,
        out_shape=(
            jax.ShapeDtypeStruct((_N, _D), jnp.float32),
            jax.ShapeDtypeStruct((_N, _D), jnp.float32),
            jax.ShapeDtypeStruct((_N, 1), jnp.float32),
        ),
    )(feature, s, degp, Wg1, bg1r, Wf1, bf1r, Wf2, bf2r,
      Wm1, bm1r, Wm2, bm2r, gr, br)

    p = _scatter_partials(hw1p, src, dst)

    hw2p = pl.pallas_call(
        _k3_body,
        out_shape=jax.ShapeDtypeStruct((_N, _D), jnp.float32),
    )(p, hw1p, dinv, Wg2, bg2r)

    q = _scatter_partials(hw2p, src, dst)

    z1 = pl.pallas_call(
        _k5_body,
        out_shape=jax.ShapeDtypeStruct((_N, _D), jnp.float32),
    )(q, hw2p, dinv)

    return (z1, z2)


# speedup vs baseline: 10.4660x; 3.4839x over previous
import functools

import jax
import jax.numpy as jnp
from jax import lax
from jax.experimental import pallas as pl
from jax.experimental.pallas import tpu as pltpu
from jax.experimental.pallas import tpu_sc as plsc

_N = 10000
_E = 320000
_D = 128

_NC = 2
_NS = 16
_NW = _NC * _NS
_K = 128
_CPT = 80
_EP = _NW * _CPT * _K
_ND = 10240
_RPT = _ND // _NS
_DUMMY = 10016

_mesh = plsc.VectorSubcoreMesh(core_axis_name="c", subcore_axis_name="s")


def _deg_body(dsts, out, acc, zb, didx, ones_v):
    c = lax.axis_index("c")
    s = lax.axis_index("s")
    wid = c * _NS + s
    zeros = jnp.zeros((16,), jnp.float32)
    ones = jnp.ones((16,), jnp.float32)

    def zloop(i, _):
        zb[pl.ds(i * 16, 16)] = zeros
        return 0
    lax.fori_loop(0, 40, zloop, 0)

    def oloop(i, _):
        ones_v[pl.ds(i * 16, 16)] = ones
        return 0
    lax.fori_loop(0, _K // 16, oloop, 0)

    pltpu.sync_copy(zb, acc.at[pl.ds(s * 640, 640)])
    pltpu.sync_copy(dsts.at[pl.ds(wid * _CPT, _CPT)], didx)
    plsc.subcore_barrier()

    def chunk(t, _):
        pltpu.sync_copy(ones_v, acc.at[didx.at[t]], add=True)
        return 0
    lax.fori_loop(0, _CPT, chunk, 0)
    plsc.subcore_barrier()
    pltpu.sync_copy(acc.at[pl.ds(s * 640, 640)], out.at[c, pl.ds(s * 640, 640)])


@functools.partial(
    pl.kernel,
    out_type=jax.ShapeDtypeStruct((_NC, _ND), jnp.float32),
    mesh=_mesh,
    scratch_types=[
        pltpu.VMEM_SHARED((_ND,), jnp.float32),
        pltpu.VMEM((640,), jnp.float32),
        pltpu.VMEM((_CPT, _K), jnp.int32),
        pltpu.VMEM((_K,), jnp.float32),
    ],
)
def _deg_sc(dsts, out, acc, zb, didx, ones_v):
    _deg_body(dsts, out, acc, zb, didx, ones_v)


def _scat_body(hw, pidx, out, acc, pk, sidx2, didx2, rows, g0, g1):
    c = lax.axis_index("c")
    s = lax.axis_index("s")
    wid = c * _NS + s
    zeros = jnp.zeros((16,), jnp.float32)
    mask = jnp.full((16,), 0xFFFF, jnp.int32)

    def zrow(i, _):
        for j in range(8):
            rows[0, i, pl.ds(j * 16, 16)] = zeros
        return 0
    lax.fori_loop(0, 128, zrow, 0)
    for t in range(5):
        pltpu.sync_copy(rows.at[0], acc.at[pl.ds(s * _RPT + t * 128, 128)])
    pltpu.sync_copy(pidx.at[pl.ds(wid * _CPT, _CPT)], pk)
    plsc.subcore_barrier()

    def unpack_src(t, slot):
        def u(j, _):
            x = pk[t, pl.ds(j * 16, 16)]
            sidx2[slot, pl.ds(j * 16, 16)] = lax.bitwise_and(x, mask)
            return 0
        lax.fori_loop(0, 8, u, 0)

    def unpack_dst(t):
        def u(j, _):
            x = pk[t, pl.ds(j * 16, 16)]
            didx2[0, pl.ds(j * 16, 16)] = lax.shift_right_logical(x, 16)
            return 0
        lax.fori_loop(0, 8, u, 0)

    gsem = [g0, g1]

    def start_gather(t, b):
        pltpu.async_copy(hw.at[sidx2.at[b]], rows.at[b], gsem[b])

    unpack_src(0, 0)
    start_gather(0, 0)
    unpack_src(1, 1)
    start_gather(1, 1)

    def pair(g, _):
        for b in range(2):
            t = g * 2 + b
            pltpu.make_async_copy(hw.at[sidx2.at[b]], rows.at[b], gsem[b]).wait()
            unpack_dst(t)
            pltpu.sync_copy(rows.at[b], acc.at[didx2.at[0]], add=True)

            @pl.when(t < _CPT - 2)
            def _():
                unpack_src(t + 2, b)
                start_gather(t + 2, b)
        return 0
    lax.fori_loop(0, _CPT // 2, pair, 0)
    plsc.subcore_barrier()
    pltpu.sync_copy(acc.at[pl.ds(s * _RPT, _RPT)],
                    out.at[c, pl.ds(s * _RPT, _RPT)])


@functools.partial(
    pl.kernel,
    out_type=jax.ShapeDtypeStruct((_NC, _ND, _D), jnp.float32),
    mesh=_mesh,
    scratch_types=[
        pltpu.VMEM_SHARED((_ND, _D), jnp.float32),
        pltpu.VMEM((_CPT, _K), jnp.int32),
        pltpu.VMEM((2, _K), jnp.int32),
        pltpu.VMEM((2, _K), jnp.int32),
        pltpu.VMEM((2, _K, _D), jnp.float32),
        pltpu.SemaphoreType.DMA,
        pltpu.SemaphoreType.DMA,
    ],
)
def _scat_sc(hw, pidx, out, acc, pk, sidx2, didx2, rows, g0, g1):
    _scat_body(hw, pidx, out, acc, pk, sidx2, didx2, rows, g0, g1)


def _k1_body(f_ref, s_ref, degp_ref, wg1_ref, bg1_ref, wf1_ref, bf1_ref,
             wf2_ref, bf2_ref, wm1_ref, bm1_ref, wm2_ref, bm2_ref,
             g_ref, b_ref, hw1_ref, z2_ref, dinv_ref):
    deg = degp_ref[0] + degp_ref[1] + 1.0
    dinv = lax.rsqrt(deg)
    dinv_ref[...] = dinv
    f = f_ref[...]
    hw1 = jnp.dot(f, wg1_ref[...], preferred_element_type=jnp.float32) + bg1_ref[...]
    hw1_ref[...] = hw1 * dinv
    t = jnp.dot(f, wf1_ref[...], preferred_element_type=jnp.float32) + bf1_ref[...]
    t = jnp.where(t > 0, t, jnp.exp(jnp.minimum(t, 0.0)) - 1.0)
    zf2 = jnp.dot(t, wf2_ref[...], preferred_element_type=jnp.float32) + bf2_ref[...]
    m = jnp.maximum(
        jnp.dot(s_ref[...], wm1_ref[...], preferred_element_type=jnp.float32)
        + bm1_ref[...], 0.0)
    m = jnp.dot(m, wm2_ref[...], preferred_element_type=jnp.float32) + bm2_ref[...]
    mu = jnp.mean(m)
    var = jnp.mean((m - mu) ** 2)
    mh = (m - mu) * lax.rsqrt(var + 1e-5) * g_ref[0, 0] + b_ref[0, 0]
    z2_ref[...] = zf2 * (1.0 / (1.0 + jnp.exp(-mh)))


def _k3_body(p_ref, hw1_ref, dinv_ref, wg2_ref, bg2_ref, hw2_ref):
    dinv = dinv_ref[...]
    h = jnp.maximum((p_ref[0] + p_ref[1] + hw1_ref[...]) * dinv, 0.0)
    hw2_ref[...] = (jnp.dot(h, wg2_ref[...], preferred_element_type=jnp.float32)
                    + bg2_ref[...]) * dinv


def _k5_body(q_ref, hw2_ref, dinv_ref, z1_ref):
    z1_ref[...] = (q_ref[0] + q_ref[1] + hw2_ref[...]) * dinv_ref[...]


def kernel(feature, edge_index, alpha, beta, Wg1, bg1, Wg2, bg2,
           Wf1, bf1, Wf2, bf2, Wm1, bm1, Wm2, bm2, bn_gamma, bn_beta):
    pad = _EP - _E
    srcp = jnp.concatenate([edge_index[0], jnp.zeros((pad,), jnp.int32)])
    dstp = jnp.concatenate([edge_index[1], jnp.full((pad,), _DUMMY, jnp.int32)])
    packed = (srcp | (dstp << 16)).reshape(_NW * _CPT, _K)
    dsts2 = dstp.reshape(_NW * _CPT, _K)
    s = jnp.stack([alpha, beta], axis=1)
    bg1r = bg1.reshape(1, -1)
    bg2r = bg2.reshape(1, -1)
    bf1r = bf1.reshape(1, -1)
    bf2r = bf2.reshape(1, -1)
    bm1r = bm1.reshape(1, -1)
    bm2r = bm2.reshape(1, -1)
    gr = bn_gamma.reshape(1, 1)
    br = bn_beta.reshape(1, 1)

    degp = _deg_sc(dsts2)[:, :_N, None]

    hw1p, z2, dinv = pl.pallas_call(
        _k1_body,
        out_shape=(
            jax.ShapeDtypeStruct((_N, _D), jnp.float32),
            jax.ShapeDtypeStruct((_N, _D), jnp.float32),
            jax.ShapeDtypeStruct((_N, 1), jnp.float32),
        ),
    )(feature, s, degp, Wg1, bg1r, Wf1, bf1r, Wf2, bf2r,
      Wm1, bm1r, Wm2, bm2r, gr, br)

    p = _scat_sc(hw1p, packed)[:, :_N]

    hw2p = pl.pallas_call(
        _k3_body,
        out_shape=jax.ShapeDtypeStruct((_N, _D), jnp.float32),
    )(p, hw1p, dinv, Wg2, bg2r)

    q = _scat_sc(hw2p, packed)[:, :_N]

    z1 = pl.pallas_call(
        _k5_body,
        out_shape=jax.ShapeDtypeStruct((_N, _D), jnp.float32),
    )(q, hw2p, dinv)

    return (z1, z2)


# speedup vs baseline: 10.5028x; 1.0035x over previous
import functools

import jax
import jax.numpy as jnp
from jax import lax
from jax.experimental import pallas as pl
from jax.experimental.pallas import tpu as pltpu
from jax.experimental.pallas import tpu_sc as plsc

_N = 10000
_E = 320000
_D = 128

_NC = 2
_NS = 16
_NW = _NC * _NS
_K = 128
_CPT = 80
_EP = _NW * _CPT * _K
_ND = 10240
_RPT = _ND // _NS
_DUMMY = 10016

_mesh = plsc.VectorSubcoreMesh(core_axis_name="c", subcore_axis_name="s")


def _deg_body(dsts, out, acc, zb, didx, ones_v):
    c = lax.axis_index("c")
    s = lax.axis_index("s")
    wid = c * _NS + s
    zeros = jnp.zeros((16,), jnp.float32)
    ones = jnp.ones((16,), jnp.float32)

    def zloop(i, _):
        zb[pl.ds(i * 16, 16)] = zeros
        return 0
    lax.fori_loop(0, 40, zloop, 0)

    def oloop(i, _):
        ones_v[pl.ds(i * 16, 16)] = ones
        return 0
    lax.fori_loop(0, _K // 16, oloop, 0)

    pltpu.sync_copy(zb, acc.at[pl.ds(s * 640, 640)])
    pltpu.sync_copy(dsts.at[pl.ds(wid * _CPT, _CPT)], didx)
    plsc.subcore_barrier()

    def chunk(t, _):
        pltpu.sync_copy(ones_v, acc.at[didx.at[t]], add=True)
        return 0
    lax.fori_loop(0, _CPT, chunk, 0)
    plsc.subcore_barrier()
    pltpu.sync_copy(acc.at[pl.ds(s * 640, 640)], out.at[c, pl.ds(s * 640, 640)])


@functools.partial(
    pl.kernel,
    out_type=jax.ShapeDtypeStruct((_NC, _ND), jnp.float32),
    mesh=_mesh,
    scratch_types=[
        pltpu.VMEM_SHARED((_ND,), jnp.float32),
        pltpu.VMEM((640,), jnp.float32),
        pltpu.VMEM((_CPT, _K), jnp.int32),
        pltpu.VMEM((_K,), jnp.float32),
    ],
)
def _deg_sc(dsts, out, acc, zb, didx, ones_v):
    _deg_body(dsts, out, acc, zb, didx, ones_v)


def _scat_body(hw, pidx, out, acc, pk, sidx2, didx2, rows, g0, g1):
    c = lax.axis_index("c")
    s = lax.axis_index("s")
    wid = c * _NS + s
    zeros = jnp.zeros((16,), jnp.float32)
    mask = jnp.full((16,), 0xFFFF, jnp.int32)

    def zrow(i, _):
        for j in range(8):
            rows[0, i, pl.ds(j * 16, 16)] = zeros
        return 0
    lax.fori_loop(0, 128, zrow, 0)
    for t in range(5):
        pltpu.sync_copy(rows.at[0], acc.at[pl.ds(s * _RPT + t * 128, 128)])
    pltpu.sync_copy(pidx.at[pl.ds(wid * _CPT, _CPT)], pk)
    plsc.subcore_barrier()

    def unpack_src(t, slot):
        def u(j, _):
            x = pk[t, pl.ds(j * 16, 16)]
            sidx2[slot, pl.ds(j * 16, 16)] = lax.bitwise_and(x, mask)
            return 0
        lax.fori_loop(0, 8, u, 0)

    def unpack_dst(t):
        def u(j, _):
            x = pk[t, pl.ds(j * 16, 16)]
            didx2[0, pl.ds(j * 16, 16)] = lax.shift_right_logical(x, 16)
            return 0
        lax.fori_loop(0, 8, u, 0)

    gsem = [g0, g1]

    def start_gather(t, b):
        pltpu.async_copy(hw.at[sidx2.at[b]], rows.at[b], gsem[b])

    unpack_src(0, 0)
    start_gather(0, 0)
    unpack_src(1, 1)
    start_gather(1, 1)

    def pair(g, _):
        for b in range(2):
            t = g * 2 + b
            pltpu.make_async_copy(hw.at[sidx2.at[b]], rows.at[b], gsem[b]).wait()
            unpack_dst(t)
            pltpu.sync_copy(rows.at[b], acc.at[didx2.at[0]], add=True)

            @pl.when(t < _CPT - 2)
            def _():
                unpack_src(t + 2, b)
                start_gather(t + 2, b)
        return 0
    lax.fori_loop(0, _CPT // 2, pair, 0)
    plsc.subcore_barrier()
    pltpu.sync_copy(acc.at[pl.ds(s * _RPT, _RPT)],
                    out.at[c, pl.ds(s * _RPT, _RPT)])


@functools.partial(
    pl.kernel,
    out_type=jax.ShapeDtypeStruct((_NC, _ND, _D), jnp.float32),
    mesh=_mesh,
    scratch_types=[
        pltpu.VMEM_SHARED((_ND, _D), jnp.float32),
        pltpu.VMEM((_CPT, _K), jnp.int32),
        pltpu.VMEM((2, _K), jnp.int32),
        pltpu.VMEM((2, _K), jnp.int32),
        pltpu.VMEM((2, _K, _D), jnp.float32),
        pltpu.SemaphoreType.DMA,
        pltpu.SemaphoreType.DMA,
    ],
)
def _scat_sc(hw, pidx, out, acc, pk, sidx2, didx2, rows, g0, g1):
    _scat_body(hw, pidx, out, acc, pk, sidx2, didx2, rows, g0, g1)


def _k1_body(f_ref, s_ref, degp_ref, wg1_ref, bg1_ref, wf1_ref, bf1_ref,
             wf2_ref, bf2_ref, wm1_ref, bm1_ref, wm2_ref, bm2_ref,
             g_ref, b_ref, hw1_ref, z2_ref, dinv_ref):
    deg = degp_ref[0] + degp_ref[1] + 1.0
    dinv = lax.rsqrt(deg)
    dinv_ref[...] = dinv
    f = f_ref[...]
    hw1 = jnp.dot(f, wg1_ref[...], preferred_element_type=jnp.float32) + bg1_ref[...]
    hw1_ref[...] = hw1 * dinv
    t = jnp.dot(f, wf1_ref[...], preferred_element_type=jnp.float32) + bf1_ref[...]
    t = jnp.where(t > 0, t, jnp.exp(jnp.minimum(t, 0.0)) - 1.0)
    zf2 = jnp.dot(t, wf2_ref[...], preferred_element_type=jnp.float32) + bf2_ref[...]
    m = jnp.maximum(
        jnp.dot(s_ref[...], wm1_ref[...], preferred_element_type=jnp.float32)
        + bm1_ref[...], 0.0)
    m = jnp.dot(m, wm2_ref[...], preferred_element_type=jnp.float32) + bm2_ref[...]
    mu = jnp.mean(m)
    var = jnp.mean((m - mu) ** 2)
    mh = (m - mu) * lax.rsqrt(var + 1e-5) * g_ref[0, 0] + b_ref[0, 0]
    z2_ref[...] = zf2 * (1.0 / (1.0 + jnp.exp(-mh)))


def _k3_body(p_ref, hw1_ref, dinv_ref, wg2_ref, bg2_ref, hw2_ref):
    dinv = dinv_ref[...]
    h = jnp.maximum((p_ref[0] + p_ref[1] + hw1_ref[...]) * dinv, 0.0)
    hw2_ref[...] = (jnp.dot(h, wg2_ref[...], preferred_element_type=jnp.float32)
                    + bg2_ref[...]) * dinv


def _k5_body(q_ref, hw2_ref, dinv_ref, z1_ref):
    z1_ref[...] = (q_ref[0] + q_ref[1] + hw2_ref[...]) * dinv_ref[...]


def kernel(feature, edge_index, alpha, beta, Wg1, bg1, Wg2, bg2,
           Wf1, bf1, Wf2, bf2, Wm1, bm1, Wm2, bm2, bn_gamma, bn_beta):
    pad = _EP - _E
    padd = _N + (jnp.arange(pad, dtype=jnp.int32) % (_ND - _N))
    srcp = jnp.concatenate([edge_index[0], jnp.zeros((pad,), jnp.int32)])
    dstp = jnp.concatenate([edge_index[1], padd])
    packed = (srcp | (dstp << 16)).reshape(_NW * _CPT, _K)
    dsts2 = dstp.reshape(_NW * _CPT, _K)
    s = jnp.stack([alpha, beta], axis=1)
    bg1r = bg1.reshape(1, -1)
    bg2r = bg2.reshape(1, -1)
    bf1r = bf1.reshape(1, -1)
    bf2r = bf2.reshape(1, -1)
    bm1r = bm1.reshape(1, -1)
    bm2r = bm2.reshape(1, -1)
    gr = bn_gamma.reshape(1, 1)
    br = bn_beta.reshape(1, 1)

    degp = _deg_sc(dsts2)[:, :_N, None]

    hw1p, z2, dinv = pl.pallas_call(
        _k1_body,
        out_shape=(
            jax.ShapeDtypeStruct((_N, _D), jnp.float32),
            jax.ShapeDtypeStruct((_N, _D), jnp.float32),
            jax.ShapeDtypeStruct((_N, 1), jnp.float32),
        ),
    )(feature, s, degp, Wg1, bg1r, Wf1, bf1r, Wf2, bf2r,
      Wm1, bm1r, Wm2, bm2r, gr, br)

    p = _scat_sc(hw1p, packed)[:, :_N]

    hw2p = pl.pallas_call(
        _k3_body,
        out_shape=jax.ShapeDtypeStruct((_N, _D), jnp.float32),
    )(p, hw1p, dinv, Wg2, bg2r)

    q = _scat_sc(hw2p, packed)[:, :_N]

    z1 = pl.pallas_call(
        _k5_body,
        out_shape=jax.ShapeDtypeStruct((_N, _D), jnp.float32),
    )(q, hw2p, dinv)

    return (z1, z2)


# speedup vs baseline: 31.4072x; 2.9904x over previous
import functools

import jax
import jax.numpy as jnp
from jax import lax
from jax.experimental import pallas as pl
from jax.experimental.pallas import tpu as pltpu
from jax.experimental.pallas import tpu_sc as plsc

_N = 10000
_E = 320000
_D = 128

_NC = 2
_NS = 16
_NW = _NC * _NS
_K = 128
_CPT = 80
_EP = _NW * _CPT * _K
_ND = 10240
_RPT = _ND // _NS
_DUMMY = 10016

_mesh = plsc.VectorSubcoreMesh(core_axis_name="c", subcore_axis_name="s")


def _deg_body(dsts, out, acc, zb, didx, ones_v):
    c = lax.axis_index("c")
    s = lax.axis_index("s")
    wid = c * _NS + s
    zeros = jnp.zeros((16,), jnp.float32)
    ones = jnp.ones((16,), jnp.float32)

    def zloop(i, _):
        zb[pl.ds(i * 16, 16)] = zeros
        return 0
    lax.fori_loop(0, 40, zloop, 0)

    def oloop(i, _):
        ones_v[pl.ds(i * 16, 16)] = ones
        return 0
    lax.fori_loop(0, _K // 16, oloop, 0)

    pltpu.sync_copy(zb, acc.at[pl.ds(s * 640, 640)])
    pltpu.sync_copy(dsts.at[pl.ds(wid * _CPT, _CPT)], didx)
    plsc.subcore_barrier()

    def chunk(t, _):
        pltpu.sync_copy(ones_v, acc.at[didx.at[t]], add=True)
        return 0
    lax.fori_loop(0, _CPT, chunk, 0)
    plsc.subcore_barrier()
    pltpu.sync_copy(acc.at[pl.ds(s * 640, 640)], out.at[c, pl.ds(s * 640, 640)])


@functools.partial(
    pl.kernel,
    out_type=jax.ShapeDtypeStruct((_NC, _ND), jnp.float32),
    mesh=_mesh,
    scratch_types=[
        pltpu.VMEM_SHARED((_ND,), jnp.float32),
        pltpu.VMEM((640,), jnp.float32),
        pltpu.VMEM((_CPT, _K), jnp.int32),
        pltpu.VMEM((_K,), jnp.float32),
    ],
)
def _deg_sc(dsts, out, acc, zb, didx, ones_v):
    _deg_body(dsts, out, acc, zb, didx, ones_v)


def _scat_body(hw, pidx, out, acc, pk, sidx2, didx2, rows, g0, g1):
    c = lax.axis_index("c")
    s = lax.axis_index("s")
    wid = c * _NS + s
    zeros = jnp.zeros((16,), jnp.float32)
    mask = jnp.full((16,), 0xFFFF, jnp.int32)

    def zrow(i, _):
        for j in range(8):
            rows[0, i, pl.ds(j * 16, 16)] = zeros
        return 0
    lax.fori_loop(0, 128, zrow, 0)
    for t in range(5):
        pltpu.sync_copy(rows.at[0], acc.at[pl.ds(s * _RPT + t * 128, 128)])
    pltpu.sync_copy(pidx.at[pl.ds(wid * _CPT, _CPT)], pk)
    plsc.subcore_barrier()

    def unpack_src(t, slot):
        def u(j, _):
            x = pk[t, pl.ds(j * 16, 16)]
            sidx2[slot, pl.ds(j * 16, 16)] = lax.bitwise_and(x, mask)
            return 0
        lax.fori_loop(0, 8, u, 0)

    def unpack_dst(t):
        def u(j, _):
            x = pk[t, pl.ds(j * 16, 16)]
            didx2[0, pl.ds(j * 16, 16)] = lax.shift_right_logical(x, 16)
            return 0
        lax.fori_loop(0, 8, u, 0)

    gsem = [g0, g1]

    def start_gather(t, b):
        pltpu.async_copy(hw.at[sidx2.at[b]], rows.at[b], gsem[b])

    unpack_src(0, 0)
    start_gather(0, 0)
    unpack_src(1, 1)
    start_gather(1, 1)

    def pair(g, _):
        for b in range(2):
            t = g * 2 + b
            pltpu.make_async_copy(hw.at[sidx2.at[b]], rows.at[b], gsem[b]).wait()
            unpack_dst(t)
            pltpu.sync_copy(rows.at[b], acc.at[didx2.at[0]], add=True)

            @pl.when(t < _CPT - 2)
            def _():
                unpack_src(t + 2, b)
                start_gather(t + 2, b)
        return 0
    lax.fori_loop(0, _CPT // 2, pair, 0)
    plsc.subcore_barrier()
    pltpu.sync_copy(acc.at[pl.ds(s * _RPT, _RPT)],
                    out.at[c, pl.ds(s * _RPT, _RPT)])


@functools.partial(
    pl.kernel,
    out_type=jax.ShapeDtypeStruct((_NC, _ND, _D), jnp.float32),
    mesh=_mesh,
    scratch_types=[
        pltpu.VMEM_SHARED((_ND, _D), jnp.float32),
        pltpu.VMEM((_CPT, _K), jnp.int32),
        pltpu.VMEM((2, _K), jnp.int32),
        pltpu.VMEM((2, _K), jnp.int32),
        pltpu.VMEM((2, _K, _D), jnp.float32),
        pltpu.SemaphoreType.DMA,
        pltpu.SemaphoreType.DMA,
    ],
)
def _scat_sc(hw, pidx, out, acc, pk, sidx2, didx2, rows, g0, g1):
    _scat_body(hw, pidx, out, acc, pk, sidx2, didx2, rows, g0, g1)


def _k1_body(f_ref, s_ref, degp_ref, wg1_ref, bg1_ref, wf1_ref, bf1_ref,
             wf2_ref, bf2_ref, wm1_ref, bm1_ref, wm2_ref, bm2_ref,
             g_ref, b_ref, hw1_ref, z2_ref, dinv_ref):
    deg = degp_ref[0] + degp_ref[1] + 1.0
    dinv = lax.rsqrt(deg)
    dinv_ref[...] = dinv
    f = f_ref[...]
    hw1 = jnp.dot(f, wg1_ref[...], preferred_element_type=jnp.float32) + bg1_ref[...]
    hw1_ref[...] = hw1 * dinv
    t = jnp.dot(f, wf1_ref[...], preferred_element_type=jnp.float32) + bf1_ref[...]
    t = jnp.where(t > 0, t, jnp.exp(jnp.minimum(t, 0.0)) - 1.0)
    zf2 = jnp.dot(t, wf2_ref[...], preferred_element_type=jnp.float32) + bf2_ref[...]
    m = jnp.maximum(
        jnp.dot(s_ref[...], wm1_ref[...], preferred_element_type=jnp.float32)
        + bm1_ref[...], 0.0)
    m = jnp.dot(m, wm2_ref[...], preferred_element_type=jnp.float32) + bm2_ref[...]
    mu = jnp.mean(m)
    var = jnp.mean((m - mu) ** 2)
    mh = (m - mu) * lax.rsqrt(var + 1e-5) * g_ref[0, 0] + b_ref[0, 0]
    z2_ref[...] = zf2 * (1.0 / (1.0 + jnp.exp(-mh)))


def _k3_body(p_ref, hw1_ref, dinv_ref, wg2_ref, bg2_ref, hw2_ref):
    dinv = dinv_ref[...]
    h = jnp.maximum((p_ref[0] + p_ref[1] + hw1_ref[...]) * dinv, 0.0)
    hw2_ref[...] = (jnp.dot(h, wg2_ref[...], preferred_element_type=jnp.float32)
                    + bg2_ref[...]) * dinv


def _k5_body(q_ref, hw2_ref, dinv_ref, z1_ref):
    z1_ref[...] = (q_ref[0] + q_ref[1] + hw2_ref[...]) * dinv_ref[...]


def kernel(feature, edge_index, alpha, beta, Wg1, bg1, Wg2, bg2,
           Wf1, bf1, Wf2, bf2, Wm1, bm1, Wm2, bm2, bn_gamma, bn_beta):
    pad = _EP - _E
    padi = jnp.arange(pad, dtype=jnp.int32)
    srcp = jnp.concatenate([edge_index[0], padi % _N])
    dstp = jnp.concatenate([edge_index[1], _N + padi % (_ND - _N)])
    packed = (srcp | (dstp << 16)).reshape(_NW * _CPT, _K)
    dsts2 = dstp.reshape(_NW * _CPT, _K)
    s = jnp.stack([alpha, beta], axis=1)
    bg1r = bg1.reshape(1, -1)
    bg2r = bg2.reshape(1, -1)
    bf1r = bf1.reshape(1, -1)
    bf2r = bf2.reshape(1, -1)
    bm1r = bm1.reshape(1, -1)
    bm2r = bm2.reshape(1, -1)
    gr = bn_gamma.reshape(1, 1)
    br = bn_beta.reshape(1, 1)

    degp = _deg_sc(dsts2)[:, :_N, None]

    hw1p, z2, dinv = pl.pallas_call(
        _k1_body,
        out_shape=(
            jax.ShapeDtypeStruct((_N, _D), jnp.float32),
            jax.ShapeDtypeStruct((_N, _D), jnp.float32),
            jax.ShapeDtypeStruct((_N, 1), jnp.float32),
        ),
    )(feature, s, degp, Wg1, bg1r, Wf1, bf1r, Wf2, bf2r,
      Wm1, bm1r, Wm2, bm2r, gr, br)

    p = _scat_sc(hw1p, packed)[:, :_N]

    hw2p = pl.pallas_call(
        _k3_body,
        out_shape=jax.ShapeDtypeStruct((_N, _D), jnp.float32),
    )(p, hw1p, dinv, Wg2, bg2r)

    q = _scat_sc(hw2p, packed)[:, :_N]

    z1 = pl.pallas_call(
        _k5_body,
        out_shape=jax.ShapeDtypeStruct((_N, _D), jnp.float32),
    )(q, hw2p, dinv)

    return (z1, z2)


# speedup vs baseline: 32.0565x; 1.0207x over previous
import functools

import jax
import jax.numpy as jnp
from jax import lax
from jax.experimental import pallas as pl
from jax.experimental.pallas import tpu as pltpu
from jax.experimental.pallas import tpu_sc as plsc

_N = 10000
_E = 320000
_D = 128

_NC = 2
_NS = 16
_NW = _NC * _NS
_K = 128
_CPT = 80
_ND = 10240
_RPT = _ND // _NS
_ER = _E // _K
_PR = _NW * _CPT
_RT = _ER - 31 * _CPT

_mesh = plsc.VectorSubcoreMesh(core_axis_name="c", subcore_axis_name="s")


def _deg_body(srcs, dsts, out, pko, acc, zb, sblk, dblk, pkb, ones_v):
    c = lax.axis_index("c")
    s = lax.axis_index("s")
    wid = c * _NS + s
    zeros = jnp.zeros((16,), jnp.float32)
    ones = jnp.ones((16,), jnp.float32)

    def zloop(i, _):
        zb[pl.ds(i * 16, 16)] = zeros
        return 0
    lax.fori_loop(0, 40, zloop, 0)

    def oloop(i, _):
        ones_v[pl.ds(i * 16, 16)] = ones
        return 0
    lax.fori_loop(0, _K // 16, oloop, 0)

    pltpu.sync_copy(zb, acc.at[pl.ds(s * 640, 640)])

    pltpu.sync_copy(srcs.at[pl.ds(wid * _CPT, _CPT)], sblk)
    pltpu.sync_copy(dsts.at[pl.ds(wid * _CPT, _CPT)], dblk)

    def packrow(t, _):
        for j in range(8):
            sv = sblk[t, pl.ds(j * 16, 16)]
            dv = dblk[t, pl.ds(j * 16, 16)]
            pkb[t, pl.ds(j * 16, 16)] = lax.bitwise_or(
                sv, lax.shift_left(dv, 16))
        return 0
    lax.fori_loop(0, _CPT, packrow, 0)
    pltpu.sync_copy(pkb, pko.at[pl.ds(wid * _CPT, _CPT)])

    plsc.subcore_barrier()

    def chunk(t, _):
        pltpu.sync_copy(ones_v, acc.at[dblk.at[t]], add=True)
        return 0
    lax.fori_loop(0, _CPT, chunk, 0)
    plsc.subcore_barrier()
    pltpu.sync_copy(acc.at[pl.ds(s * 640, 640)], out.at[c, pl.ds(s * 640, 640)])


@functools.partial(
    pl.kernel,
    out_type=(
        jax.ShapeDtypeStruct((_NC, _ND), jnp.float32),
        jax.ShapeDtypeStruct((_PR, _K), jnp.int32),
    ),
    mesh=_mesh,
    scratch_types=[
        pltpu.VMEM_SHARED((_ND,), jnp.float32),
        pltpu.VMEM((640,), jnp.float32),
        pltpu.VMEM((_CPT, _K), jnp.int32),
        pltpu.VMEM((_CPT, _K), jnp.int32),
        pltpu.VMEM((_CPT, _K), jnp.int32),
        pltpu.VMEM((_K,), jnp.float32),
    ],
)
def _deg_sc(srcs, dsts, out, pko, acc, zb, sblk, dblk, pkb, ones_v):
    _deg_body(srcs, dsts, out, pko, acc, zb, sblk, dblk, pkb, ones_v)


def _scat_body(hw, pidx, out, acc, pk, sidx2, didx2, rows, g0, g1):
    c = lax.axis_index("c")
    s = lax.axis_index("s")
    wid = c * _NS + s
    zeros = jnp.zeros((16,), jnp.float32)
    mask = jnp.full((16,), 0xFFFF, jnp.int32)

    def zrow(i, _):
        for j in range(8):
            rows[0, i, pl.ds(j * 16, 16)] = zeros
        return 0
    lax.fori_loop(0, 128, zrow, 0)
    for t in range(5):
        pltpu.sync_copy(rows.at[0], acc.at[pl.ds(s * _RPT + t * 128, 128)])
    pltpu.sync_copy(pidx.at[pl.ds(wid * _CPT, _CPT)], pk)
    plsc.subcore_barrier()

    def unpack_src(t, slot):
        def u(j, _):
            x = pk[t, pl.ds(j * 16, 16)]
            sidx2[slot, pl.ds(j * 16, 16)] = lax.bitwise_and(x, mask)
            return 0
        lax.fori_loop(0, 8, u, 0)

    def unpack_dst(t):
        def u(j, _):
            x = pk[t, pl.ds(j * 16, 16)]
            didx2[0, pl.ds(j * 16, 16)] = lax.shift_right_logical(x, 16)
            return 0
        lax.fori_loop(0, 8, u, 0)

    gsem = [g0, g1]

    def start_gather(t, b):
        pltpu.async_copy(hw.at[sidx2.at[b]], rows.at[b], gsem[b])

    unpack_src(0, 0)
    start_gather(0, 0)
    unpack_src(1, 1)
    start_gather(1, 1)

    def pair(g, _):
        for b in range(2):
            t = g * 2 + b
            pltpu.make_async_copy(hw.at[sidx2.at[b]], rows.at[b], gsem[b]).wait()
            unpack_dst(t)
            pltpu.sync_copy(rows.at[b], acc.at[didx2.at[0]], add=True)

            @pl.when(t < _CPT - 2)
            def _():
                unpack_src(t + 2, b)
                start_gather(t + 2, b)
        return 0
    lax.fori_loop(0, _CPT // 2, pair, 0)
    plsc.subcore_barrier()
    pltpu.sync_copy(acc.at[pl.ds(s * _RPT, _RPT)],
                    out.at[c, pl.ds(s * _RPT, _RPT)])


@functools.partial(
    pl.kernel,
    out_type=jax.ShapeDtypeStruct((_NC, _ND, _D), jnp.float32),
    mesh=_mesh,
    scratch_types=[
        pltpu.VMEM_SHARED((_ND, _D), jnp.float32),
        pltpu.VMEM((_CPT, _K), jnp.int32),
        pltpu.VMEM((2, _K), jnp.int32),
        pltpu.VMEM((2, _K), jnp.int32),
        pltpu.VMEM((2, _K, _D), jnp.float32),
        pltpu.SemaphoreType.DMA,
        pltpu.SemaphoreType.DMA,
    ],
)
def _scat_sc(hw, pidx, out, acc, pk, sidx2, didx2, rows, g0, g1):
    _scat_body(hw, pidx, out, acc, pk, sidx2, didx2, rows, g0, g1)


def _k1a_body(f_ref, s_ref, wg1_ref, bg1_ref, wf1_ref, bf1_ref,
              wf2_ref, bf2_ref, wm1_ref, bm1_ref, wm2_ref, bm2_ref,
              hw1_ref, zf2_ref, m_ref):
    f = f_ref[...]
    hw1_ref[...] = jnp.dot(f, wg1_ref[...],
                           preferred_element_type=jnp.float32) + bg1_ref[...]
    t = jnp.dot(f, wf1_ref[...], preferred_element_type=jnp.float32) + bf1_ref[...]
    t = jnp.where(t > 0, t, jnp.exp(jnp.minimum(t, 0.0)) - 1.0)
    zf2_ref[...] = jnp.dot(t, wf2_ref[...],
                           preferred_element_type=jnp.float32) + bf2_ref[...]
    m = jnp.maximum(
        jnp.dot(s_ref[...], wm1_ref[...], preferred_element_type=jnp.float32)
        + bm1_ref[...], 0.0)
    m_ref[...] = jnp.dot(m, wm2_ref[...],
                         preferred_element_type=jnp.float32) + bm2_ref[...]


def _k1b_body(degp_ref, hw1_ref, hw1p_ref, dinv_ref):
    deg = degp_ref[0] + degp_ref[1] + 1.0
    dinv = lax.rsqrt(deg)
    dinv_ref[...] = dinv
    hw1p_ref[...] = hw1_ref[...] * dinv


def _k1c_body(zf2_ref, m_ref, g_ref, b_ref, z2_ref):
    m = m_ref[...]
    mu = jnp.mean(m)
    var = jnp.mean((m - mu) ** 2)
    mh = (m - mu) * lax.rsqrt(var + 1e-5) * g_ref[0, 0] + b_ref[0, 0]
    z2_ref[...] = zf2_ref[...] * (1.0 / (1.0 + jnp.exp(-mh)))


def _k3_body(p_ref, hw1_ref, dinv_ref, wg2_ref, bg2_ref, hw2_ref):
    dinv = dinv_ref[...]
    h = jnp.maximum((p_ref[0] + p_ref[1] + hw1_ref[...]) * dinv, 0.0)
    hw2_ref[...] = (jnp.dot(h, wg2_ref[...], preferred_element_type=jnp.float32)
                    + bg2_ref[...]) * dinv


def _k5_body(q_ref, hw2_ref, dinv_ref, z1_ref):
    z1_ref[...] = (q_ref[0] + q_ref[1] + hw2_ref[...]) * dinv_ref[...]


def kernel(feature, edge_index, alpha, beta, Wg1, bg1, Wg2, bg2,
           Wf1, bf1, Wf2, bf2, Wm1, bm1, Wm2, bm2, bn_gamma, bn_beta):
    pad = _PR * _K - _E
    padi = jnp.arange(pad, dtype=jnp.int32)
    srcs = jnp.concatenate([edge_index[0], padi % _N]).reshape(_PR, _K)
    dsts = jnp.concatenate([edge_index[1], _N + padi % (_ND - _N)]).reshape(_PR, _K)
    s = jnp.stack([alpha, beta], axis=1)
    bg1r = bg1.reshape(1, -1)
    bg2r = bg2.reshape(1, -1)
    bf1r = bf1.reshape(1, -1)
    bf2r = bf2.reshape(1, -1)
    bm1r = bm1.reshape(1, -1)
    bm2r = bm2.reshape(1, -1)
    gr = bn_gamma.reshape(1, 1)
    br = bn_beta.reshape(1, 1)

    degp_raw, packed = _deg_sc(srcs, dsts)
    degp = degp_raw[:, :_N, None]

    hw1_raw, zf2, m_raw = pl.pallas_call(
        _k1a_body,
        out_shape=(
            jax.ShapeDtypeStruct((_N, _D), jnp.float32),
            jax.ShapeDtypeStruct((_N, _D), jnp.float32),
            jax.ShapeDtypeStruct((_N, 1), jnp.float32),
        ),
    )(feature, s, Wg1, bg1r, Wf1, bf1r, Wf2, bf2r, Wm1, bm1r, Wm2, bm2r)

    hw1p, dinv = pl.pallas_call(
        _k1b_body,
        out_shape=(
            jax.ShapeDtypeStruct((_N, _D), jnp.float32),
            jax.ShapeDtypeStruct((_N, 1), jnp.float32),
        ),
    )(degp, hw1_raw)

    z2 = pl.pallas_call(
        _k1c_body,
        out_shape=jax.ShapeDtypeStruct((_N, _D), jnp.float32),
    )(zf2, m_raw, gr, br)

    p = _scat_sc(hw1p, packed)[:, :_N]

    hw2p = pl.pallas_call(
        _k3_body,
        out_shape=jax.ShapeDtypeStruct((_N, _D), jnp.float32),
    )(p, hw1p, dinv, Wg2, bg2r)

    q = _scat_sc(hw2p, packed)[:, :_N]

    z1 = pl.pallas_call(
        _k5_body,
        out_shape=jax.ShapeDtypeStruct((_N, _D), jnp.float32),
    )(q, hw2p, dinv)

    return (z1, z2)


# speedup vs baseline: 32.2493x; 1.0060x over previous
import functools

import jax
import jax.numpy as jnp
from jax import lax
from jax.experimental import pallas as pl
from jax.experimental.pallas import tpu as pltpu
from jax.experimental.pallas import tpu_sc as plsc

_N = 10000
_E = 320000
_D = 128

_NC = 2
_NS = 16
_NW = _NC * _NS
_K = 128
_CPT = 80
_ND = 10112
_RPT = _ND // _NS
_NDH = 10240
_RPH = _NDH // _NS
_PR = _NW * _CPT
_K2 = 80
_CPT2 = 128

_mesh = plsc.VectorSubcoreMesh(core_axis_name="c", subcore_axis_name="s")


def _deg_body(srcs, dsts, out, pko, acc, zb, sblk, dblk, pkb, ones_v):
    c = lax.axis_index("c")
    s = lax.axis_index("s")
    wid = c * _NS + s
    zeros = jnp.zeros((16,), jnp.float32)
    ones = jnp.ones((16,), jnp.float32)

    def zloop(i, _):
        zb[pl.ds(i * 16, 16)] = zeros
        return 0
    lax.fori_loop(0, 40, zloop, 0)

    def oloop(i, _):
        ones_v[pl.ds(i * 16, 16)] = ones
        return 0
    lax.fori_loop(0, _K // 16, oloop, 0)

    pltpu.sync_copy(zb, acc.at[pl.ds(s * _RPH, _RPH)])

    pltpu.sync_copy(srcs.at[pl.ds(wid * _CPT, _CPT)], sblk)
    pltpu.sync_copy(dsts.at[pl.ds(wid * _CPT, _CPT)], dblk)

    def packblk(blk, _):
        for u in range(40):
            ts, js = (blk * 5) + (u // 8), (u % 8) * 16
            tp, jp = (blk * 8) + (u // 5), (u % 5) * 16
            sv = sblk[ts, pl.ds(js, 16)]
            dv = dblk[ts, pl.ds(js, 16)]
            pkb[tp, pl.ds(jp, 16)] = lax.bitwise_or(
                sv, lax.shift_left(dv, 16))
        return 0
    lax.fori_loop(0, 16, packblk, 0)
    pltpu.sync_copy(pkb, pko.at[pl.ds(wid * _CPT2, _CPT2)])

    plsc.subcore_barrier()

    def chunk(t, _):
        pltpu.sync_copy(ones_v, acc.at[dblk.at[t]], add=True)
        return 0
    lax.fori_loop(0, _CPT, chunk, 0)
    plsc.subcore_barrier()
    pltpu.sync_copy(acc.at[pl.ds(s * _RPH, _RPH)],
                    out.at[c, pl.ds(s * _RPH, _RPH)])


@functools.partial(
    pl.kernel,
    out_type=(
        jax.ShapeDtypeStruct((_NC, _NDH), jnp.float32),
        jax.ShapeDtypeStruct((_NW * _CPT2, _K2), jnp.int32),
    ),
    mesh=_mesh,
    scratch_types=[
        pltpu.VMEM_SHARED((_NDH,), jnp.float32),
        pltpu.VMEM((_RPH,), jnp.float32),
        pltpu.VMEM((_CPT, _K), jnp.int32),
        pltpu.VMEM((_CPT, _K), jnp.int32),
        pltpu.VMEM((_CPT2, _K2), jnp.int32),
        pltpu.VMEM((_K,), jnp.float32),
    ],
)
def _deg_sc(srcs, dsts, out, pko, acc, zb, sblk, dblk, pkb, ones_v):
    _deg_body(srcs, dsts, out, pko, acc, zb, sblk, dblk, pkb, ones_v)


def _scat_body(hw, pidx, out, acc, pk, idxb, zb, rows, g0, g1, g2, s0, s1):
    c = lax.axis_index("c")
    s = lax.axis_index("s")
    wid = c * _NS + s
    zeros = jnp.zeros((16,), jnp.float32)
    mask = jnp.full((16,), 0xFFFF, jnp.int32)

    def zrow(i, _):
        for j in range(8):
            zb[i, pl.ds(j * 16, 16)] = zeros
        return 0
    lax.fori_loop(0, 8, zrow, 0)

    def zcp(i, _):
        pltpu.sync_copy(zb, acc.at[pl.ds(s * _RPT + i * 8, 8)])
        return 0
    lax.fori_loop(0, _RPT // 8, zcp, 0)
    pltpu.sync_copy(pidx.at[pl.ds(wid * _CPT2, _CPT2)], pk)
    plsc.subcore_barrier()

    def unpack_src(t, slot):
        def u(j, _):
            x = pk[t, pl.ds(j * 16, 16)]
            idxb[slot, pl.ds(j * 16, 16)] = lax.bitwise_and(x, mask)
            return 0
        lax.fori_loop(0, _K2 // 16, u, 0)

    def unpack_dst(t, slot):
        def u(j, _):
            x = pk[t, pl.ds(j * 16, 16)]
            idxb[2 + slot, pl.ds(j * 16, 16)] = lax.shift_right_logical(x, 16)
            return 0
        lax.fori_loop(0, _K2 // 16, u, 0)

    gsem = [g0, g1, g2]
    ssem = [s0, s1]

    def start_gather(rb, islot):
        pltpu.async_copy(hw.at[idxb.at[islot]], rows.at[rb], gsem[rb])

    unpack_src(0, 0)
    start_gather(0, 0)
    unpack_src(1, 1)
    start_gather(1, 1)

    def step(t, rb, pa, do_prefetch):
        pltpu.make_async_copy(hw.at[idxb.at[pa]], rows.at[rb], gsem[rb]).wait()
        unpack_dst(t, pa)
        pltpu.async_copy(rows.at[rb], acc.at[idxb.at[2 + pa]], ssem[pa],
                         add=True)
        if do_prefetch:
            @pl.when(t + 2 < _CPT2)
            def _():
                @pl.when(t >= 1)
                def _():
                    pltpu.make_async_copy(
                        rows.at[0], acc.at[idxb.at[2 + (1 - pa)]],
                        ssem[1 - pa]).wait()
                unpack_src(t + 2, pa)
                start_gather((rb + 2) % 3, pa)

    def six(g, _):
        for b in range(6):
            t6 = g * 6 + b
            step(t6, b % 3, b % 2, True)
        return 0
    lax.fori_loop(0, _CPT2 // 6, six, 0)
    for b in range(_CPT2 % 6):
        t6 = (_CPT2 // 6) * 6 + b
        step(t6, b % 3, b % 2, False)
    for pa in (1, 0, 1):
        pltpu.make_async_copy(rows.at[0], acc.at[idxb.at[2 + pa]],
                              ssem[pa]).wait()
    plsc.subcore_barrier()
    pltpu.sync_copy(acc.at[pl.ds(s * _RPT, _RPT)],
                    out.at[c, pl.ds(s * _RPT, _RPT)])


@functools.partial(
    pl.kernel,
    out_type=jax.ShapeDtypeStruct((_NC, _ND, _D), jnp.float32),
    mesh=_mesh,
    scratch_types=[
        pltpu.VMEM_SHARED((_ND, _D), jnp.float32),
        pltpu.VMEM((_CPT2, _K2), jnp.int32),
        pltpu.VMEM((8, _K2), jnp.int32),
        pltpu.VMEM((8, _D), jnp.float32),
        pltpu.VMEM((3, _K2, _D), jnp.float32),
        pltpu.SemaphoreType.DMA,
        pltpu.SemaphoreType.DMA,
        pltpu.SemaphoreType.DMA,
        pltpu.SemaphoreType.DMA,
        pltpu.SemaphoreType.DMA,
    ],
)
def _scat_sc(hw, pidx, out, acc, pk, idxb, zb, rows, g0, g1, g2, s0, s1):
    _scat_body(hw, pidx, out, acc, pk, idxb, zb, rows, g0, g1, g2, s0, s1)


def _k1a_body(f_ref, s_ref, wg1_ref, bg1_ref, wf1_ref, bf1_ref,
              wf2_ref, bf2_ref, wm1_ref, bm1_ref, wm2_ref, bm2_ref,
              hw1_ref, zf2_ref, m_ref):
    f = f_ref[...]
    hw1_ref[...] = jnp.dot(f, wg1_ref[...],
                           preferred_element_type=jnp.float32) + bg1_ref[...]
    t = jnp.dot(f, wf1_ref[...], preferred_element_type=jnp.float32) + bf1_ref[...]
    t = jnp.where(t > 0, t, jnp.exp(jnp.minimum(t, 0.0)) - 1.0)
    zf2_ref[...] = jnp.dot(t, wf2_ref[...],
                           preferred_element_type=jnp.float32) + bf2_ref[...]
    m = jnp.maximum(
        jnp.dot(s_ref[...], wm1_ref[...], preferred_element_type=jnp.float32)
        + bm1_ref[...], 0.0)
    m_ref[...] = jnp.dot(m, wm2_ref[...],
                         preferred_element_type=jnp.float32) + bm2_ref[...]


def _k1b_body(degp_ref, hw1_ref, hw1p_ref, dinv_ref):
    deg = degp_ref[0] + degp_ref[1] + 1.0
    dinv = lax.rsqrt(deg)
    dinv_ref[...] = dinv
    hw1p_ref[...] = hw1_ref[...] * dinv


def _k1c_body(zf2_ref, m_ref, g_ref, b_ref, z2_ref):
    m = m_ref[...]
    mu = jnp.mean(m)
    var = jnp.mean((m - mu) ** 2)
    mh = (m - mu) * lax.rsqrt(var + 1e-5) * g_ref[0, 0] + b_ref[0, 0]
    z2_ref[...] = zf2_ref[...] * (1.0 / (1.0 + jnp.exp(-mh)))


def _k3_body(p_ref, hw1_ref, dinv_ref, wg2_ref, bg2_ref, hw2_ref):
    dinv = dinv_ref[...]
    h = jnp.maximum((p_ref[0] + p_ref[1] + hw1_ref[...]) * dinv, 0.0)
    hw2_ref[...] = (jnp.dot(h, wg2_ref[...], preferred_element_type=jnp.float32)
                    + bg2_ref[...]) * dinv


def _k5_body(q_ref, hw2_ref, dinv_ref, z1_ref):
    z1_ref[...] = (q_ref[0] + q_ref[1] + hw2_ref[...]) * dinv_ref[...]


def kernel(feature, edge_index, alpha, beta, Wg1, bg1, Wg2, bg2,
           Wf1, bf1, Wf2, bf2, Wm1, bm1, Wm2, bm2, bn_gamma, bn_beta):
    pad = _PR * _K - _E
    padi = jnp.arange(pad, dtype=jnp.int32)
    srcs = jnp.concatenate([edge_index[0], padi % _N]).reshape(_PR, _K)
    dsts = jnp.concatenate([edge_index[1], _N + padi % (_ND - _N)]).reshape(_PR, _K)
    s = jnp.stack([alpha, beta], axis=1)
    bg1r = bg1.reshape(1, -1)
    bg2r = bg2.reshape(1, -1)
    bf1r = bf1.reshape(1, -1)
    bf2r = bf2.reshape(1, -1)
    bm1r = bm1.reshape(1, -1)
    bm2r = bm2.reshape(1, -1)
    gr = bn_gamma.reshape(1, 1)
    br = bn_beta.reshape(1, 1)

    degp_raw, packed = _deg_sc(srcs, dsts)
    degp = degp_raw[:, :_N, None]

    hw1_raw, zf2, m_raw = pl.pallas_call(
        _k1a_body,
        out_shape=(
            jax.ShapeDtypeStruct((_N, _D), jnp.float32),
            jax.ShapeDtypeStruct((_N, _D), jnp.float32),
            jax.ShapeDtypeStruct((_N, 1), jnp.float32),
        ),
    )(feature, s, Wg1, bg1r, Wf1, bf1r, Wf2, bf2r, Wm1, bm1r, Wm2, bm2r)

    hw1p, dinv = pl.pallas_call(
        _k1b_body,
        out_shape=(
            jax.ShapeDtypeStruct((_N, _D), jnp.float32),
            jax.ShapeDtypeStruct((_N, 1), jnp.float32),
        ),
    )(degp, hw1_raw)

    z2 = pl.pallas_call(
        _k1c_body,
        out_shape=jax.ShapeDtypeStruct((_N, _D), jnp.float32),
    )(zf2, m_raw, gr, br)

    p = _scat_sc(hw1p, packed)[:, :_N]

    hw2p = pl.pallas_call(
        _k3_body,
        out_shape=jax.ShapeDtypeStruct((_N, _D), jnp.float32),
    )(p, hw1p, dinv, Wg2, bg2r)

    q = _scat_sc(hw2p, packed)[:, :_N]

    z1 = pl.pallas_call(
        _k5_body,
        out_shape=jax.ShapeDtypeStruct((_N, _D), jnp.float32),
    )(q, hw2p, dinv)

    return (z1, z2)


# speedup vs baseline: 33.7495x; 1.0465x over previous
import functools

import jax
import jax.numpy as jnp
from jax import lax
from jax.experimental import pallas as pl
from jax.experimental.pallas import tpu as pltpu
from jax.experimental.pallas import tpu_sc as plsc

_N = 10000
_E = 320000
_D = 128

_NC = 2
_NS = 16
_NW = _NC * _NS
_K = 128
_CPT = 80
_ND = 10112
_RPT = _ND // _NS
_NDH = 10240
_RPH = _NDH // _NS
_PR = _NW * _CPT
_K2 = 80
_CPT2 = 128

_mesh = plsc.VectorSubcoreMesh(core_axis_name="c", subcore_axis_name="s")


def _deg_body(srcs, dsts, out, pko, acc, zb, sblk, dblk, pkb, ones_v):
    c = lax.axis_index("c")
    s = lax.axis_index("s")
    wid = c * _NS + s
    zeros = jnp.zeros((16,), jnp.float32)
    ones = jnp.ones((16,), jnp.float32)

    def zloop(i, _):
        zb[pl.ds(i * 16, 16)] = zeros
        return 0
    lax.fori_loop(0, 40, zloop, 0)

    def oloop(i, _):
        ones_v[pl.ds(i * 16, 16)] = ones
        return 0
    lax.fori_loop(0, _K // 16, oloop, 0)

    pltpu.sync_copy(zb, acc.at[pl.ds(s * _RPH, _RPH)])

    pltpu.sync_copy(srcs.at[pl.ds(wid * _CPT, _CPT)], sblk)
    pltpu.sync_copy(dsts.at[pl.ds(wid * _CPT, _CPT)], dblk)

    def packblk(blk, _):
        for u in range(40):
            ts, js = (blk * 5) + (u // 8), (u % 8) * 16
            tp, jp = (blk * 8) + (u // 5), (u % 5) * 16
            sv = sblk[ts, pl.ds(js, 16)]
            dv = dblk[ts, pl.ds(js, 16)]
            pkb[tp, pl.ds(jp, 16)] = lax.bitwise_or(
                sv, lax.shift_left(dv, 16))
        return 0
    lax.fori_loop(0, 16, packblk, 0)
    pltpu.sync_copy(pkb, pko.at[pl.ds(wid * _CPT2, _CPT2)])

    plsc.subcore_barrier()

    def chunk(t, _):
        pltpu.sync_copy(ones_v, acc.at[dblk.at[t]], add=True)
        return 0
    lax.fori_loop(0, _CPT, chunk, 0)
    plsc.subcore_barrier()
    pltpu.sync_copy(acc.at[pl.ds(s * _RPH, _RPH)],
                    out.at[c, pl.ds(s * _RPH, _RPH)])


@functools.partial(
    pl.kernel,
    out_type=(
        jax.ShapeDtypeStruct((_NC, _NDH), jnp.float32),
        jax.ShapeDtypeStruct((_NW * _CPT2, _K2), jnp.int32),
    ),
    mesh=_mesh,
    scratch_types=[
        pltpu.VMEM_SHARED((_NDH,), jnp.float32),
        pltpu.VMEM((_RPH,), jnp.float32),
        pltpu.VMEM((_CPT, _K), jnp.int32),
        pltpu.VMEM((_CPT, _K), jnp.int32),
        pltpu.VMEM((_CPT2, _K2), jnp.int32),
        pltpu.VMEM((_K,), jnp.float32),
    ],
)
def _deg_sc(srcs, dsts, out, pko, acc, zb, sblk, dblk, pkb, ones_v):
    _deg_body(srcs, dsts, out, pko, acc, zb, sblk, dblk, pkb, ones_v)


def _scat_body(hw, pidx, out, acc, pk, idxb, zb, rows, g0, g1, g2, s0, s1):
    c = lax.axis_index("c")
    s = lax.axis_index("s")
    wid = c * _NS + s
    zeros = jnp.zeros((16,), jnp.float32)
    mask = jnp.full((16,), 0xFFFF, jnp.int32)

    def zrow(i, _):
        for j in range(8):
            zb[i, pl.ds(j * 16, 16)] = zeros
        return 0
    lax.fori_loop(0, 8, zrow, 0)

    def zcp(i, _):
        pltpu.sync_copy(zb, acc.at[pl.ds(s * _RPT + i * 8, 8)])
        return 0
    lax.fori_loop(0, _RPT // 8, zcp, 0)
    pltpu.sync_copy(pidx.at[pl.ds(wid * _CPT2, _CPT2)], pk)
    plsc.subcore_barrier()

    def unpack_src(t, slot):
        def u(j, _):
            x = pk[t, pl.ds(j * 16, 16)]
            idxb[slot, pl.ds(j * 16, 16)] = lax.bitwise_and(x, mask)
            return 0
        lax.fori_loop(0, _K2 // 16, u, 0)

    def unpack_dst(t, slot):
        def u(j, _):
            x = pk[t, pl.ds(j * 16, 16)]
            idxb[2 + slot, pl.ds(j * 16, 16)] = lax.shift_right_logical(x, 16)
            return 0
        lax.fori_loop(0, _K2 // 16, u, 0)

    gsem = [g0, g1, g2]
    ssem = [s0, s1]

    def start_gather(rb, islot):
        pltpu.async_copy(hw.at[idxb.at[islot]], rows.at[rb], gsem[rb])

    unpack_src(0, 0)
    start_gather(0, 0)
    unpack_src(1, 1)
    start_gather(1, 1)

    def step(t, rb, pa, do_prefetch):
        pltpu.make_async_copy(hw.at[idxb.at[pa]], rows.at[rb], gsem[rb]).wait()
        unpack_dst(t, pa)
        pltpu.async_copy(rows.at[rb], acc.at[idxb.at[2 + pa]], ssem[pa],
                         add=True)
        if do_prefetch:
            @pl.when(t + 2 < _CPT2)
            def _():
                @pl.when(t >= 1)
                def _():
                    pltpu.make_async_copy(
                        rows.at[0], acc.at[idxb.at[2 + (1 - pa)]],
                        ssem[1 - pa]).wait()
                unpack_src(t + 2, pa)
                start_gather((rb + 2) % 3, pa)

    def six(g, _):
        for b in range(6):
            t6 = g * 6 + b
            step(t6, b % 3, b % 2, True)
        return 0
    lax.fori_loop(0, _CPT2 // 6, six, 0)
    for b in range(_CPT2 % 6):
        t6 = (_CPT2 // 6) * 6 + b
        step(t6, b % 3, b % 2, False)
    for pa in (1, 0, 1):
        pltpu.make_async_copy(rows.at[0], acc.at[idxb.at[2 + pa]],
                              ssem[pa]).wait()
    plsc.subcore_barrier()
    pltpu.sync_copy(acc.at[pl.ds(s * _RPT, _RPT)],
                    out.at[c, pl.ds(s * _RPT, _RPT)])


@functools.partial(
    pl.kernel,
    out_type=jax.ShapeDtypeStruct((_NC, _ND, _D), jnp.float32),
    mesh=_mesh,
    scratch_types=[
        pltpu.VMEM_SHARED((_ND, _D), jnp.float32),
        pltpu.VMEM((_CPT2, _K2), jnp.int32),
        pltpu.VMEM((8, _K2), jnp.int32),
        pltpu.VMEM((8, _D), jnp.float32),
        pltpu.VMEM((3, _K2, _D), jnp.float32),
        pltpu.SemaphoreType.DMA,
        pltpu.SemaphoreType.DMA,
        pltpu.SemaphoreType.DMA,
        pltpu.SemaphoreType.DMA,
        pltpu.SemaphoreType.DMA,
    ],
)
def _scat_sc(hw, pidx, out, acc, pk, idxb, zb, rows, g0, g1, g2, s0, s1):
    _scat_body(hw, pidx, out, acc, pk, idxb, zb, rows, g0, g1, g2, s0, s1)


def _k1a_body(f_ref, s_ref, wg1_ref, bg1_ref, wf1_ref, bf1_ref,
              wf2_ref, bf2_ref, wm1_ref, bm1_ref, wm2_ref, bm2_ref,
              hw1_ref, zf2_ref, m_ref):
    f = f_ref[...]
    hw1_ref[...] = jnp.dot(f, wg1_ref[...],
                           preferred_element_type=jnp.float32) + bg1_ref[...]
    t = jnp.dot(f, wf1_ref[...], preferred_element_type=jnp.float32) + bf1_ref[...]
    t = jnp.where(t > 0, t, jnp.exp(jnp.minimum(t, 0.0)) - 1.0)
    zf2_ref[...] = jnp.dot(t, wf2_ref[...],
                           preferred_element_type=jnp.float32) + bf2_ref[...]
    m = jnp.maximum(
        jnp.dot(s_ref[...], wm1_ref[...], preferred_element_type=jnp.float32)
        + bm1_ref[...], 0.0)
    m_ref[...] = jnp.dot(m, wm2_ref[...],
                         preferred_element_type=jnp.float32) + bm2_ref[...]


def _k1b_body(degp_ref, hw1_ref, hw1p_ref, dinv_ref):
    deg = degp_ref[0] + degp_ref[1] + 1.0
    dinv = lax.rsqrt(deg)
    dinv_ref[...] = dinv
    hw1p_ref[...] = hw1_ref[...] * dinv


def _k1c_body(zf2_ref, m_ref, g_ref, b_ref, z2_ref):
    m = m_ref[...]
    mu = jnp.mean(m)
    var = jnp.mean((m - mu) ** 2)
    mh = (m - mu) * lax.rsqrt(var + 1e-5) * g_ref[0, 0] + b_ref[0, 0]
    z2_ref[...] = zf2_ref[...] * (1.0 / (1.0 + jnp.exp(-mh)))


def _k3_body(p_ref, hw1_ref, dinv_ref, wg2_ref, bg2_ref, hw2_ref):
    dinv = dinv_ref[...]
    h = jnp.maximum(
        (p_ref[0, :_N] + p_ref[1, :_N] + hw1_ref[...]) * dinv, 0.0)
    hw2_ref[...] = (jnp.dot(h, wg2_ref[...], preferred_element_type=jnp.float32)
                    + bg2_ref[...]) * dinv


def _k5_body(q_ref, hw2_ref, dinv_ref, z1_ref):
    z1_ref[...] = (q_ref[0, :_N] + q_ref[1, :_N]
                   + hw2_ref[...]) * dinv_ref[...]


def kernel(feature, edge_index, alpha, beta, Wg1, bg1, Wg2, bg2,
           Wf1, bf1, Wf2, bf2, Wm1, bm1, Wm2, bm2, bn_gamma, bn_beta):
    pad = _PR * _K - _E
    padi = jnp.arange(pad, dtype=jnp.int32)
    srcs = jnp.concatenate([edge_index[0], padi % _N]).reshape(_PR, _K)
    dsts = jnp.concatenate([edge_index[1], _N + padi % (_ND - _N)]).reshape(_PR, _K)
    s = jnp.stack([alpha, beta], axis=1)
    bg1r = bg1.reshape(1, -1)
    bg2r = bg2.reshape(1, -1)
    bf1r = bf1.reshape(1, -1)
    bf2r = bf2.reshape(1, -1)
    bm1r = bm1.reshape(1, -1)
    bm2r = bm2.reshape(1, -1)
    gr = bn_gamma.reshape(1, 1)
    br = bn_beta.reshape(1, 1)

    degp_raw, packed = _deg_sc(srcs, dsts)
    degp = degp_raw[:, :_N, None]

    hw1_raw, zf2, m_raw = pl.pallas_call(
        _k1a_body,
        out_shape=(
            jax.ShapeDtypeStruct((_N, _D), jnp.float32),
            jax.ShapeDtypeStruct((_N, _D), jnp.float32),
            jax.ShapeDtypeStruct((_N, 1), jnp.float32),
        ),
    )(feature, s, Wg1, bg1r, Wf1, bf1r, Wf2, bf2r, Wm1, bm1r, Wm2, bm2r)

    hw1p, dinv = pl.pallas_call(
        _k1b_body,
        out_shape=(
            jax.ShapeDtypeStruct((_N, _D), jnp.float32),
            jax.ShapeDtypeStruct((_N, 1), jnp.float32),
        ),
    )(degp, hw1_raw)

    z2 = pl.pallas_call(
        _k1c_body,
        out_shape=jax.ShapeDtypeStruct((_N, _D), jnp.float32),
    )(zf2, m_raw, gr, br)

    p = _scat_sc(hw1p, packed)

    hw2p = pl.pallas_call(
        _k3_body,
        out_shape=jax.ShapeDtypeStruct((_N, _D), jnp.float32),
    )(p, hw1p, dinv, Wg2, bg2r)

    q = _scat_sc(hw2p, packed)

    z1 = pl.pallas_call(
        _k5_body,
        out_shape=jax.ShapeDtypeStruct((_N, _D), jnp.float32),
    )(q, hw2p, dinv)

    return (z1, z2)
